# baseline (speedup 1.0000x reference)
import jax
import jax.numpy as jnp
import numpy as np
from jax import lax
from jax.experimental import pallas as pl
from jax.experimental.pallas import tpu as pltpu
from jax.experimental.pallas import tpu_sc as plsc

B = 4096
D = 64
T = 5
NC_TOT = 25
CB = 32
NCHUNK = B // CB
NWORK = 32
CPW = NCHUNK // NWORK

S_OFF, A_OFF, P_OFF, N_OFF = 32, 192, 352, 512
ROWS = 832
TI_LEN = 352
TO_LEN = 480
TI_BLOCKS = [(0, 128), (128, 128), (256, 96)]
TO_BLOCKS = [(0, 128), (128, 128), (256, 128), (384, 96)]
NBLK = len(TI_BLOCKS) + len(TO_BLOCKS)
DOTS = CB * NC_TOT

_J = np.arange(DOTS)
_B = _J // NC_TOT
_C = _J % NC_TOT
_ROWTAB = np.where(
    _C < 5, P_OFF + _B * 5 + _C,
    np.where(_C < 15, N_OFF + _B * 10 + (_C - 5),
             np.where(_C < 20, S_OFF + _B * 5 + (_C - 15),
                      A_OFF + _B * 5 + (_C - 20)))).astype(np.int32)
_ROWTAB = _ROWTAB.reshape(DOTS // 16, 16)
_BTAB = _B.astype(np.int32).reshape(DOTS // 16, 16)
_RIDX = np.arange(ROWS, dtype=np.int32).reshape(ROWS // 16, 16)


def _sc_body(ti, to, w2, p2, n2, s2, a2, rt, bt, ridx, out,
             raw, tix_v, oix_v, rt_v, bt_v, ridx_v, acc, dotbuf,
             sem_idx, semw1, sem_add, semo):
    nc = 2
    wid = lax.axis_index("s") * nc + lax.axis_index("c")
    pltpu.sync_copy(rt, rt_v)
    pltpu.sync_copy(bt, bt_v)
    pltpu.sync_copy(ridx, ridx_v)

    def issue_raw(c, s):
        g = wid * CPW + c
        return [
            pltpu.async_copy(w2.at[pl.ds(g * CB, CB)],
                             raw.at[s, pl.ds(0, 32)], sem_idx),
            pltpu.async_copy(s2.at[pl.ds(g * 160, 160)],
                             raw.at[s, pl.ds(32, 160)], sem_idx),
            pltpu.async_copy(a2.at[pl.ds(g * 160, 160)],
                             raw.at[s, pl.ds(192, 160)], sem_idx),
            pltpu.async_copy(p2.at[pl.ds(g * 160, 160)],
                             raw.at[s, pl.ds(352, 160)], sem_idx),
            pltpu.async_copy(n2.at[pl.ds(g * 320, 320)],
                             raw.at[s, pl.ds(512, 320)], sem_idx),
        ]

    def extract(s):
        for t in range(T):
            tcol = jnp.full((16,), t, jnp.int32)

            def ex_ti(grp, carry):
                vals = plsc.load_gather(raw.at[s], [ridx_v[grp], tcol])
                tix_v[t, pl.ds(grp * 16, 16)] = vals
                return carry

            def ex_to(grp, carry):
                vals = plsc.load_gather(raw.at[s], [ridx_v[22 + grp], tcol])
                oix_v[t, pl.ds(grp * 16, 16)] = vals
                return carry

            lax.fori_loop(0, TI_LEN // 16, ex_ti, 0)
            lax.fori_loop(0, TO_LEN // 16, ex_to, 0)

    def block_copy(s, t, i, sem, add):
        if i < len(TI_BLOCKS):
            off, ln = TI_BLOCKS[i]
            return pltpu.async_copy(
                ti.at[tix_v.at[t, pl.ds(off, ln)]],
                acc.at[s, pl.ds(off, ln)], sem, add=add)
        off, ln = TO_BLOCKS[i - len(TI_BLOCKS)]
        return pltpu.async_copy(
            to.at[oix_v.at[t, pl.ds(off, ln)]],
            acc.at[s, pl.ds(TI_LEN + off, ln)], sem, add=add)

    def issue_wave1(s):
        return [block_copy(s, 0, i, semw1.at[i], add=False)
                for i in range(NBLK)]

    def dots(s):
        def grp_body(grp, carry2):
            row = rt_v[grp]
            b = bt_v[grp]
            dot = jnp.zeros((16,), jnp.float32)
            for d in range(D):
                dcol = jnp.full((16,), d, jnp.int32)
                ctx = plsc.load_gather(acc.at[s], [row, dcol])
                inp = plsc.load_gather(acc.at[s], [b, dcol])
                dot = dot + ctx * inp
            dotbuf[s, pl.ds(grp * 16, 16)] = dot
            return carry2

        lax.fori_loop(0, DOTS // 16, grp_body, 0)

    out_cps = [None] * CPW
    raw_cps = issue_raw(0, 0)
    for cp in raw_cps:
        cp.wait()
    extract(0)
    w1 = issue_wave1(0)
    for c in range(CPW):
        s = c & 1
        if c + 1 < CPW:
            raw_cps = issue_raw(c + 1, 1 - s)
        add_cps = []
        for i, cp in enumerate(w1):
            cp.wait()
            for t in range(1, T):
                add_cps.append(block_copy(s, t, i, sem_add, add=True))
        for cp in add_cps:
            cp.wait()
        if c + 1 < CPW:
            for cp in raw_cps:
                cp.wait()
            extract(1 - s)
            w1 = issue_wave1(1 - s)
        if c >= 2:
            out_cps[c - 2].wait()
        dots(s)
        out_cps[c] = pltpu.async_copy(
            dotbuf.at[s], out.at[wid * CPW + c], semo.at[s])
    for c in range(CPW - 2, CPW):
        out_cps[c].wait()


def _tc_body(dots_ref, ms_ref, ma_ref, out_ref):
    x = dots_ref[...]
    eps = jnp.float32(1e-10)
    col = lax.broadcasted_iota(jnp.int32, x.shape, 1)

    def sp(v):
        return jnp.maximum(v, 0.0) + jnp.log1p(jnp.exp(-jnp.abs(v)))

    sp_neg = sp(-(x + eps))
    sp_pos = sp(x - eps)
    ms = ms_ref[...]
    ma = ma_ref[...]
    zero = jnp.float32(0.0)
    p_s = jnp.sum(jnp.where(col < 5, sp_neg, zero))
    n_s = jnp.sum(jnp.where((col >= 5) & (col < 15), sp_pos, zero))
    s_s = jnp.sum(jnp.where((col >= 15) & (col < 20), sp_neg * ms, zero))
    a_s = jnp.sum(jnp.where(col >= 20, sp_pos * ma, zero))
    out_ref[0, 0] = p_s
    out_ref[0, 1] = n_s
    out_ref[0, 2] = s_s
    out_ref[0, 3] = a_s


def kernel(w_ix, p_ix, n_ix, s_ix, ms_ix, a_ix, ma_ix, table_i, table_o):
    p2 = p_ix.reshape(B * 5, T)
    n2 = n_ix.reshape(B * 10, T)
    s2 = s_ix.reshape(B * 5, T)
    a2 = a_ix.reshape(B * 5, T)

    mesh = plsc.VectorSubcoreMesh(core_axis_name="c", subcore_axis_name="s")
    sc = pl.kernel(
        _sc_body,
        out_type=jax.ShapeDtypeStruct((NCHUNK, DOTS), jnp.float32),
        mesh=mesh,
        scratch_types=[
            pltpu.VMEM((2, ROWS, T), jnp.int32),
            pltpu.VMEM((T, TI_LEN), jnp.int32),
            pltpu.VMEM((T, TO_LEN), jnp.int32),
            pltpu.VMEM((DOTS // 16, 16), jnp.int32),
            pltpu.VMEM((DOTS // 16, 16), jnp.int32),
            pltpu.VMEM((ROWS // 16, 16), jnp.int32),
            pltpu.VMEM((2, ROWS, D), jnp.float32),
            pltpu.VMEM((2, DOTS), jnp.float32),
            pltpu.SemaphoreType.DMA,
            pltpu.SemaphoreType.DMA((NBLK,)),
            pltpu.SemaphoreType.DMA,
            pltpu.SemaphoreType.DMA((2,)),
        ],
        compiler_params=pltpu.CompilerParams(use_tc_tiling_on_sc=False,
                                             needs_layout_passes=False),
    )
    dots = sc(table_i, table_o, w_ix, p2, n2, s2, a2,
              jnp.asarray(_ROWTAB), jnp.asarray(_BTAB),
              jnp.asarray(_RIDX)).reshape(B, NC_TOT)

    scores = pl.pallas_call(
        _tc_body,
        out_shape=jax.ShapeDtypeStruct((1, 4), jnp.float32),
        out_specs=pl.BlockSpec(memory_space=pltpu.SMEM),
    )(dots, ms_ix, ma_ix)

    p_s = scores[0, 0] / B
    n_s = scores[0, 1] / B
    s_s = scores[0, 2] / B
    a_s = scores[0, 3] / B
    loss = p_s + n_s + s_s + a_s
    return (loss, p_s, n_s, s_s, a_s)
